# baseline (speedup 1.0000x reference)
import functools

import jax
import jax.numpy as jnp
from jax import lax
from jax.experimental import pallas as pl
from jax.experimental.pallas import tpu as pltpu
from jax.experimental.pallas import tpu_sc as plsc

_INFO = plsc.get_sparse_core_info()
_NC = _INFO.num_cores
_NS = _INFO.num_subcores
_NW = _NC * _NS

_CHUNK = 128


def _make_gather(n_rows: int, dim: int):
    assert n_rows % (_NW * _CHUNK) == 0
    rows_per_w = n_rows // _NW
    n_chunks = rows_per_w // _CHUNK
    mesh = plsc.VectorSubcoreMesh(core_axis_name="c", subcore_axis_name="s")

    @functools.partial(
        pl.kernel,
        mesh=mesh,
        out_type=jax.ShapeDtypeStruct((n_rows, dim), jnp.float32),
        scratch_types=[
            pltpu.VMEM((_CHUNK,), jnp.int32),
            pltpu.VMEM((_CHUNK, dim), jnp.float32),
            pltpu.SemaphoreType.DMA,
        ],
        compiler_params=pltpu.CompilerParams(use_tc_tiling_on_sc=False),
    )
    def gather_kernel(idx_hbm, table_hbm, out_hbm, idx_v, rows_v, sem):
        wid = lax.axis_index("s") * _NC + lax.axis_index("c")
        base = wid * rows_per_w

        def body(i, _):
            off = base + i * _CHUNK
            pltpu.sync_copy(idx_hbm.at[pl.ds(off, _CHUNK)], idx_v)
            pltpu.async_copy(table_hbm.at[idx_v], rows_v, sem).wait()
            pltpu.sync_copy(rows_v, out_hbm.at[pl.ds(off, _CHUNK)])
            return 0

        lax.fori_loop(0, n_chunks, body, 0)

    return gather_kernel


def kernel(token_ids, weight):
    batch, hist = token_ids.shape
    _, dim = weight.shape
    flat = token_ids.reshape(-1).astype(jnp.int32)
    out = _make_gather(flat.shape[0], dim)(flat, weight)
    return out.reshape(batch, hist, dim)


# speedup vs baseline: 1.1883x; 1.1883x over previous
import functools

import jax
import jax.numpy as jnp
from jax import lax
from jax.experimental import pallas as pl
from jax.experimental.pallas import tpu as pltpu
from jax.experimental.pallas import tpu_sc as plsc

_INFO = plsc.get_sparse_core_info()
_NC = _INFO.num_cores
_NS = _INFO.num_subcores
_NW = _NC * _NS

_CHUNK = 512


def _make_gather(n_rows: int, dim: int):
    assert n_rows % _NW == 0
    rows_per_w = n_rows // _NW
    assert rows_per_w % (2 * _CHUNK) == 0
    n_pairs = rows_per_w // (2 * _CHUNK)
    mesh = plsc.VectorSubcoreMesh(core_axis_name="c", subcore_axis_name="s")

    @functools.partial(
        pl.kernel,
        mesh=mesh,
        out_type=jax.ShapeDtypeStruct((n_rows, dim), jnp.float32),
        scratch_types=[
            pltpu.VMEM((rows_per_w,), jnp.int32),
            pltpu.VMEM((_CHUNK, dim), jnp.float32),
            pltpu.VMEM((_CHUNK, dim), jnp.float32),
            pltpu.SemaphoreType.DMA,
            pltpu.SemaphoreType.DMA,
            pltpu.SemaphoreType.DMA,
            pltpu.SemaphoreType.DMA,
        ],
        compiler_params=pltpu.CompilerParams(use_tc_tiling_on_sc=False),
    )
    def gather_kernel(idx_hbm, table_hbm, out_hbm, idx_v, rows0, rows1,
                      sg0, sg1, ss0, ss1):
        wid = lax.axis_index("s") * _NC + lax.axis_index("c")
        base = wid * rows_per_w
        rows = (rows0, rows1)
        sg = (sg0, sg1)
        ss = (ss0, ss1)

        pltpu.sync_copy(idx_hbm.at[pl.ds(base, rows_per_w)], idx_v)

        @pl.loop(0, n_pairs)
        def pair(g):
            for b in range(2):
                i = g * 2 + b
                @pl.when(g > 0)
                def _():
                    pltpu.make_async_copy(
                        rows[b], out_hbm.at[pl.ds(0, _CHUNK)], ss[b]).wait()
                pltpu.async_copy(
                    table_hbm.at[idx_v.at[pl.ds(i * _CHUNK, _CHUNK)]],
                    rows[b], sg[b])
            for b in range(2):
                i = g * 2 + b
                pltpu.make_async_copy(
                    table_hbm.at[idx_v.at[pl.ds(i * _CHUNK, _CHUNK)]],
                    rows[b], sg[b]).wait()
                pltpu.async_copy(
                    rows[b], out_hbm.at[pl.ds(base + i * _CHUNK, _CHUNK)],
                    ss[b])

        for b in range(2):
            pltpu.make_async_copy(
                rows[b], out_hbm.at[pl.ds(0, _CHUNK)], ss[b]).wait()

    return gather_kernel


def kernel(token_ids, weight):
    batch, hist = token_ids.shape
    _, dim = weight.shape
    flat = token_ids.reshape(-1).astype(jnp.int32)
    out = _make_gather(flat.shape[0], dim)(flat, weight)
    return out.reshape(batch, hist, dim)


# speedup vs baseline: 1.1910x; 1.0022x over previous
import functools

import jax
import jax.numpy as jnp
from jax import lax
from jax.experimental import pallas as pl
from jax.experimental.pallas import tpu as pltpu
from jax.experimental.pallas import tpu_sc as plsc

_INFO = plsc.get_sparse_core_info()
_NC = _INFO.num_cores
_NS = _INFO.num_subcores
_NW = _NC * _NS

_NBUF = 4


def _make_gather(batch: int, hist: int, dim: int):
    assert batch % _NW == 0
    rows_per_w = batch // _NW
    assert rows_per_w % _NBUF == 0
    n_groups = rows_per_w // _NBUF
    mesh = plsc.VectorSubcoreMesh(core_axis_name="c", subcore_axis_name="s")

    @functools.partial(
        pl.kernel,
        mesh=mesh,
        out_type=jax.ShapeDtypeStruct((batch, hist, dim), jnp.float32),
        scratch_types=[
            pltpu.VMEM((rows_per_w, hist), jnp.int32),
            [pltpu.VMEM((hist, dim), jnp.float32) for _ in range(_NBUF)],
            [pltpu.SemaphoreType.DMA for _ in range(_NBUF)],
            [pltpu.SemaphoreType.DMA for _ in range(_NBUF)],
        ],
        compiler_params=pltpu.CompilerParams(use_tc_tiling_on_sc=False),
    )
    def gather_kernel(idx_hbm, table_hbm, out_hbm, idx_v, rows, sg, ss):
        wid = lax.axis_index("s") * _NC + lax.axis_index("c")
        base = wid * rows_per_w

        pltpu.sync_copy(idx_hbm.at[pl.ds(base, rows_per_w)], idx_v)

        @pl.loop(0, n_groups)
        def group(g):
            for b in range(_NBUF):
                r = g * _NBUF + b
                @pl.when(g > 0)
                def _():
                    pltpu.make_async_copy(rows[b], out_hbm.at[0], ss[b]).wait()
                pltpu.async_copy(table_hbm.at[idx_v.at[r]], rows[b], sg[b])
            for b in range(_NBUF):
                r = g * _NBUF + b
                pltpu.make_async_copy(
                    table_hbm.at[idx_v.at[r]], rows[b], sg[b]).wait()
                pltpu.async_copy(rows[b], out_hbm.at[base + r], ss[b])

        for b in range(_NBUF):
            pltpu.make_async_copy(rows[b], out_hbm.at[0], ss[b]).wait()

    return gather_kernel


def kernel(token_ids, weight):
    batch, hist = token_ids.shape
    _, dim = weight.shape
    return _make_gather(batch, hist, dim)(token_ids, weight)
